# baseline (speedup 1.0000x reference)
import dataclasses
import functools

import jax
import jax.numpy as jnp
from jax import lax
from jax.experimental import pallas as pl
from jax.experimental.pallas import tpu as pltpu
from jax.experimental.pallas import tpu_sc as plsc

N = 10000
E = 320000
D = 128

NP = 10240
BR = 1280
C = 128
NTILES = 32
NCH = 81
EPT = NCH * C
EP = NTILES * EPT
ACC_ROWS = 10112
STRIPE = ACC_ROWS // 16

_mesh = plsc.VectorSubcoreMesh(core_axis_name="c", subcore_axis_name="s")

_sc_params = pltpu.CompilerParams()
if "needs_layout_passes" in pltpu.CompilerParams.__dataclass_fields__:
    _sc_params = dataclasses.replace(_sc_params, needs_layout_passes=False)


def _deg_body(dst_hbm, zn_hbm, o_hbm, idx_v, hist_v, sem):
    cid = lax.axis_index("c")
    sid = lax.axis_index("s")
    wid = cid * 16 + sid
    pltpu.async_copy(dst_hbm.at[pl.ds(wid * EPT, EPT)], idx_v, sem).wait()
    pltpu.sync_copy(zn_hbm, hist_v)
    ones = jnp.full((16,), 1.0, jnp.float32)

    @pl.loop(0, EPT, step=16)
    def _(i):
        idx = idx_v[pl.ds(i, 16)]
        plsc.addupdate_scatter(hist_v, [idx], ones)

    pltpu.sync_copy(hist_v, o_hbm.at[wid])


def _sc_degree(dst_flat, zeros_n):
    kern = pl.kernel(
        _deg_body,
        out_type=jax.ShapeDtypeStruct((NTILES, NP), jnp.float32),
        mesh=_mesh,
        scratch_types=[
            pltpu.VMEM((EPT,), jnp.int32),
            pltpu.VMEM((NP,), jnp.float32),
            pltpu.SemaphoreType.DMA,
        ],
        compiler_params=_sc_params,
    )
    return kern(dst_flat, zeros_n)


def _scat_body(g_hbm, src_hbm, dst_hbm, zr_hbm, o_hbm,
               si0, si1, si2, di0, di1, di2, buf0, buf1, buf2, acc,
               qi0, qi1, qi2, ri0, ri1, ri2, sg0, sg1, sg2, ss0, ss1, ss2):
    cid = lax.axis_index("c")
    sid = lax.axis_index("s")
    wid = cid * 16 + sid
    base = wid * NCH
    sis = (si0, si1, si2)
    dis = (di0, di1, di2)
    bufs = (buf0, buf1, buf2)
    qis = (qi0, qi1, qi2)
    ris = (ri0, ri1, ri2)
    sgs = (sg0, sg1, sg2)
    sss = (ss0, ss1, ss2)

    for k in range(3):
        pltpu.async_copy(src_hbm.at[base + k], sis[k], qis[k])
        pltpu.async_copy(dst_hbm.at[base + k], dis[k], ris[k])
    pltpu.sync_copy(zr_hbm, acc.at[pl.ds(sid * STRIPE, STRIPE)])
    plsc.subcore_barrier()

    for k in range(3):
        pltpu.make_async_copy(src_hbm.at[base + k], sis[k], qis[k]).wait()
        pltpu.async_copy(g_hbm.at[sis[k]], bufs[k], sgs[k])

    @pl.loop(0, NCH - 3, step=3)
    def _(j):
        for k in range(3):
            pltpu.make_async_copy(g_hbm.at[sis[k]], bufs[k], sgs[k]).wait()
            pltpu.async_copy(src_hbm.at[base + j + k + 3], sis[k], qis[k])
            pltpu.make_async_copy(dst_hbm.at[base], dis[k], ris[k]).wait()
            pltpu.async_copy(bufs[k], acc.at[dis[k]], sss[k], add=True)
        for k in range(3):
            pltpu.make_async_copy(bufs[k], acc.at[dis[k]], sss[k]).wait()
            pltpu.async_copy(dst_hbm.at[base + j + k + 3], dis[k], ris[k])
            pltpu.make_async_copy(src_hbm.at[base], sis[k], qis[k]).wait()
            pltpu.async_copy(g_hbm.at[sis[k]], bufs[k], sgs[k])

    for k in range(3):
        pltpu.make_async_copy(g_hbm.at[sis[k]], bufs[k], sgs[k]).wait()
        pltpu.make_async_copy(dst_hbm.at[base], dis[k], ris[k]).wait()
        pltpu.async_copy(bufs[k], acc.at[dis[k]], sss[k], add=True)
    for k in range(3):
        pltpu.make_async_copy(bufs[k], acc.at[dis[k]], sss[k]).wait()
    plsc.subcore_barrier()

    pltpu.sync_copy(acc.at[pl.ds(sid * STRIPE, STRIPE)],
                    o_hbm.at[cid, pl.ds(sid * STRIPE, STRIPE)])


def _sc_scatter(g, src2d, dst2d, zrow):
    kern = pl.kernel(
        _scat_body,
        out_type=jax.ShapeDtypeStruct((2, NP, D), jnp.float32),
        mesh=_mesh,
        scratch_types=(
            [pltpu.VMEM((C,), jnp.int32)] * 6
            + [pltpu.VMEM((C, D), jnp.float32)] * 3
            + [pltpu.VMEM_SHARED((ACC_ROWS, D), jnp.float32)]
            + [pltpu.SemaphoreType.DMA] * 12
        ),
    )
    return kern(g, src2d, dst2d, zrow)


def _dinv_of(hist_ref):
    return lax.rsqrt(jnp.sum(hist_ref[...], axis=0) + 1.0)


def _tc1_body(hist_ref, x_ref, w_ref, g_ref):
    dinv = _dinv_of(hist_ref)
    m = jnp.dot(x_ref[...], w_ref[...], preferred_element_type=jnp.float32)
    g_ref[...] = m * dinv[:, None]


def _tc2_body(hist_ref, a_ref, g1_ref, b1_ref, w_ref, g2_ref):
    dinv = _dinv_of(hist_ref)
    h = dinv[:, None] * (a_ref[0] + a_ref[1] + g1_ref[...]) + b1_ref[...][None, :]
    h = jnp.maximum(h, 0.0)
    m = jnp.dot(h, w_ref[...], preferred_element_type=jnp.float32)
    g2_ref[...] = m * dinv[:, None]


def _tc3_body(hist_ref, a_ref, g2_ref, b2_ref, o_ref):
    dinv = _dinv_of(hist_ref)
    o_ref[...] = (dinv[:, None] * (a_ref[0] + a_ref[1] + g2_ref[...])
                  + b2_ref[...][None, :])


_HIST_SPEC = pl.BlockSpec((NTILES, BR), lambda i: (0, i))
_ROW_SPEC = pl.BlockSpec((BR, D), lambda i: (i, 0))
_ACC_SPEC = pl.BlockSpec((2, BR, D), lambda i: (0, i, 0))
_VEC_SPEC = pl.BlockSpec((D,), lambda i: (0,))
_W_SPEC = pl.BlockSpec((D, D), lambda i: (0, 0))
_OUT_ROW = jax.ShapeDtypeStruct((NP, D), jnp.float32)


def _tc1(hist, x_p, W1):
    return pl.pallas_call(
        _tc1_body,
        grid=(NP // BR,),
        in_specs=[_HIST_SPEC, _ROW_SPEC, _W_SPEC],
        out_specs=_ROW_SPEC,
        out_shape=_OUT_ROW,
    )(hist, x_p, W1)


def _tc2(hist, acc, g1, b1, W2):
    return pl.pallas_call(
        _tc2_body,
        grid=(NP // BR,),
        in_specs=[_HIST_SPEC, _ACC_SPEC, _ROW_SPEC, _VEC_SPEC, _W_SPEC],
        out_specs=_ROW_SPEC,
        out_shape=_OUT_ROW,
    )(hist, acc, g1, b1, W2)


def _tc3(hist, acc, g2, b2):
    return pl.pallas_call(
        _tc3_body,
        grid=(NP // BR,),
        in_specs=[_HIST_SPEC, _ACC_SPEC, _ROW_SPEC, _VEC_SPEC],
        out_specs=_ROW_SPEC,
        out_shape=_OUT_ROW,
    )(hist, acc, g2, b2)


def kernel(x, edge_index, W1, b1, W2, b2):
    src = edge_index[0]
    dst = edge_index[1]
    pad_i = jnp.arange(EP - E, dtype=jnp.int32)
    src_p = jnp.concatenate([src, pad_i % N])
    dst_p = jnp.concatenate([dst, N + pad_i % (ACC_ROWS - N)])
    src2d = src_p.reshape(NTILES * NCH, C)
    dst2d = dst_p.reshape(NTILES * NCH, C)
    x_p = jnp.concatenate([x, jnp.zeros((NP - N, D), jnp.float32)])
    zeros_n = jnp.zeros((NP,), jnp.float32)
    zrow = jnp.zeros((STRIPE, D), jnp.float32)

    hist = _sc_degree(dst_p, zeros_n)
    g1 = _tc1(hist, x_p, W1)
    acc1 = _sc_scatter(g1, src2d, dst2d, zrow)
    g2 = _tc2(hist, acc1, g1, b1, W2)
    acc2 = _sc_scatter(g2, src2d, dst2d, zrow)
    out = _tc3(hist, acc2, g2, b2)
    return out[:N]


# speedup vs baseline: 1.0015x; 1.0015x over previous
import dataclasses
import functools

import jax
import jax.numpy as jnp
from jax import lax
from jax.experimental import pallas as pl
from jax.experimental.pallas import tpu as pltpu
from jax.experimental.pallas import tpu_sc as plsc

N = 10000
E = 320000
D = 128

NP = 10240
BR = 1280
C = 128
NTILES = 32
NCH = 81
EPT = NCH * C
EP = NTILES * EPT
ACC_ROWS = 10112
STRIPE = ACC_ROWS // 16

_mesh = plsc.VectorSubcoreMesh(core_axis_name="c", subcore_axis_name="s")

_sc_params = pltpu.CompilerParams()
if "needs_layout_passes" in pltpu.CompilerParams.__dataclass_fields__:
    _sc_params = dataclasses.replace(_sc_params, needs_layout_passes=False)


def _deg_body(dst_hbm, zn_hbm, o_hbm, idx_v, hist_v, sem):
    cid = lax.axis_index("c")
    sid = lax.axis_index("s")
    wid = cid * 16 + sid
    pltpu.async_copy(dst_hbm.at[pl.ds(wid * EPT, EPT)], idx_v, sem).wait()
    pltpu.sync_copy(zn_hbm, hist_v)
    ones = jnp.full((16,), 1.0, jnp.float32)

    @pl.loop(0, EPT, step=64)
    def _(i):
        for u in range(4):
            idx = idx_v[pl.ds(i + u * 16, 16)]
            plsc.addupdate_scatter(hist_v, [idx], ones)

    pltpu.sync_copy(hist_v, o_hbm.at[wid])


def _sc_degree(dst_flat, zeros_n):
    kern = pl.kernel(
        _deg_body,
        out_type=jax.ShapeDtypeStruct((NTILES, NP), jnp.float32),
        mesh=_mesh,
        scratch_types=[
            pltpu.VMEM((EPT,), jnp.int32),
            pltpu.VMEM((NP,), jnp.float32),
            pltpu.SemaphoreType.DMA,
        ],
        compiler_params=_sc_params,
    )
    return kern(dst_flat, zeros_n)


def _scat_body(g_hbm, src_hbm, dst_hbm, zr_hbm, o_hbm,
               si0, si1, si2, di0, di1, di2, buf0, buf1, buf2, acc,
               qi0, qi1, qi2, ri0, ri1, ri2, sg0, sg1, sg2, ss0, ss1, ss2):
    cid = lax.axis_index("c")
    sid = lax.axis_index("s")
    wid = cid * 16 + sid
    base = wid * NCH
    sis = (si0, si1, si2)
    dis = (di0, di1, di2)
    bufs = (buf0, buf1, buf2)
    qis = (qi0, qi1, qi2)
    ris = (ri0, ri1, ri2)
    sgs = (sg0, sg1, sg2)
    sss = (ss0, ss1, ss2)

    for k in range(3):
        pltpu.async_copy(src_hbm.at[base + k], sis[k], qis[k])
        pltpu.async_copy(dst_hbm.at[base + k], dis[k], ris[k])
    pltpu.sync_copy(zr_hbm, acc.at[pl.ds(sid * STRIPE, STRIPE)])
    plsc.subcore_barrier()

    for k in range(3):
        pltpu.make_async_copy(src_hbm.at[base + k], sis[k], qis[k]).wait()
        pltpu.async_copy(g_hbm.at[sis[k]], bufs[k], sgs[k])

    @pl.loop(0, NCH - 3, step=3)
    def _(j):
        for k in range(3):
            pltpu.make_async_copy(g_hbm.at[sis[k]], bufs[k], sgs[k]).wait()
            pltpu.async_copy(src_hbm.at[base + j + k + 3], sis[k], qis[k])
            pltpu.make_async_copy(dst_hbm.at[base], dis[k], ris[k]).wait()
            pltpu.async_copy(bufs[k], acc.at[dis[k]], sss[k], add=True)
        for k in range(3):
            pltpu.make_async_copy(bufs[k], acc.at[dis[k]], sss[k]).wait()
            pltpu.async_copy(dst_hbm.at[base + j + k + 3], dis[k], ris[k])
            pltpu.make_async_copy(src_hbm.at[base], sis[k], qis[k]).wait()
            pltpu.async_copy(g_hbm.at[sis[k]], bufs[k], sgs[k])

    for k in range(3):
        pltpu.make_async_copy(g_hbm.at[sis[k]], bufs[k], sgs[k]).wait()
        pltpu.make_async_copy(dst_hbm.at[base], dis[k], ris[k]).wait()
        pltpu.async_copy(bufs[k], acc.at[dis[k]], sss[k], add=True)
    for k in range(3):
        pltpu.make_async_copy(bufs[k], acc.at[dis[k]], sss[k]).wait()
    plsc.subcore_barrier()

    pltpu.sync_copy(acc.at[pl.ds(sid * STRIPE, STRIPE)],
                    o_hbm.at[cid, pl.ds(sid * STRIPE, STRIPE)])


def _sc_scatter(g, src2d, dst2d, zrow):
    kern = pl.kernel(
        _scat_body,
        out_type=jax.ShapeDtypeStruct((2, NP, D), jnp.float32),
        mesh=_mesh,
        scratch_types=(
            [pltpu.VMEM((C,), jnp.int32)] * 6
            + [pltpu.VMEM((C, D), jnp.float32)] * 3
            + [pltpu.VMEM_SHARED((ACC_ROWS, D), jnp.float32)]
            + [pltpu.SemaphoreType.DMA] * 12
        ),
    )
    return kern(g, src2d, dst2d, zrow)


def _dinv_of(hist_ref):
    return lax.rsqrt(jnp.sum(hist_ref[...], axis=0) + 1.0)


def _tc1_body(hist_ref, x_ref, w_ref, g_ref):
    dinv = _dinv_of(hist_ref)
    m = jnp.dot(x_ref[...], w_ref[...], preferred_element_type=jnp.float32)
    g_ref[...] = m * dinv[:, None]


def _tc2_body(hist_ref, a_ref, g1_ref, b1_ref, w_ref, g2_ref):
    dinv = _dinv_of(hist_ref)
    h = dinv[:, None] * (a_ref[0] + a_ref[1] + g1_ref[...]) + b1_ref[...][None, :]
    h = jnp.maximum(h, 0.0)
    m = jnp.dot(h, w_ref[...], preferred_element_type=jnp.float32)
    g2_ref[...] = m * dinv[:, None]


def _tc3_body(hist_ref, a_ref, g2_ref, b2_ref, o_ref):
    dinv = _dinv_of(hist_ref)
    o_ref[...] = (dinv[:, None] * (a_ref[0] + a_ref[1] + g2_ref[...])
                  + b2_ref[...][None, :])


_HIST_SPEC = pl.BlockSpec((NTILES, BR), lambda i: (0, i))
_ROW_SPEC = pl.BlockSpec((BR, D), lambda i: (i, 0))
_ACC_SPEC = pl.BlockSpec((2, BR, D), lambda i: (0, i, 0))
_VEC_SPEC = pl.BlockSpec((D,), lambda i: (0,))
_W_SPEC = pl.BlockSpec((D, D), lambda i: (0, 0))
_OUT_ROW = jax.ShapeDtypeStruct((NP, D), jnp.float32)


def _tc1(hist, x_p, W1):
    return pl.pallas_call(
        _tc1_body,
        grid=(NP // BR,),
        in_specs=[_HIST_SPEC, _ROW_SPEC, _W_SPEC],
        out_specs=_ROW_SPEC,
        out_shape=_OUT_ROW,
    )(hist, x_p, W1)


def _tc2(hist, acc, g1, b1, W2):
    return pl.pallas_call(
        _tc2_body,
        grid=(NP // BR,),
        in_specs=[_HIST_SPEC, _ACC_SPEC, _ROW_SPEC, _VEC_SPEC, _W_SPEC],
        out_specs=_ROW_SPEC,
        out_shape=_OUT_ROW,
    )(hist, acc, g1, b1, W2)


def _tc3(hist, acc, g2, b2):
    return pl.pallas_call(
        _tc3_body,
        grid=(NP // BR,),
        in_specs=[_HIST_SPEC, _ACC_SPEC, _ROW_SPEC, _VEC_SPEC],
        out_specs=_ROW_SPEC,
        out_shape=_OUT_ROW,
    )(hist, acc, g2, b2)


def kernel(x, edge_index, W1, b1, W2, b2):
    src = edge_index[0]
    dst = edge_index[1]
    pad_i = jnp.arange(EP - E, dtype=jnp.int32)
    src_p = jnp.concatenate([src, pad_i % N])
    dst_p = jnp.concatenate([dst, N + pad_i % (ACC_ROWS - N)])
    src2d = src_p.reshape(NCH, NTILES, C).transpose(1, 0, 2).reshape(NTILES * NCH, C)
    dst2d = dst_p.reshape(NCH, NTILES, C).transpose(1, 0, 2).reshape(NTILES * NCH, C)
    x_p = jnp.concatenate([x, jnp.zeros((NP - N, D), jnp.float32)])
    zeros_n = jnp.zeros((NP,), jnp.float32)
    zrow = jnp.zeros((STRIPE, D), jnp.float32)

    hist = _sc_degree(dst_p, zeros_n)
    g1 = _tc1(hist, x_p, W1)
    acc1 = _sc_scatter(g1, src2d, dst2d, zrow)
    g2 = _tc2(hist, acc1, g1, b1, W2)
    acc2 = _sc_scatter(g2, src2d, dst2d, zrow)
    out = _tc3(hist, acc2, g2, b2)
    return out[:N]


# speedup vs baseline: 1.0082x; 1.0067x over previous
import dataclasses
import functools

import jax
import jax.numpy as jnp
from jax import lax
from jax.experimental import pallas as pl
from jax.experimental.pallas import tpu as pltpu
from jax.experimental.pallas import tpu_sc as plsc

N = 10000
E = 320000
D = 128

NP = 10240
BR = 2048
C = 128
NTILES = 32
NCH = 81
EPT = NCH * C
EP = NTILES * EPT
ACC_ROWS = 10112
STRIPE = ACC_ROWS // 16

_mesh = plsc.VectorSubcoreMesh(core_axis_name="c", subcore_axis_name="s")

_sc_params = pltpu.CompilerParams()
if "needs_layout_passes" in pltpu.CompilerParams.__dataclass_fields__:
    _sc_params = dataclasses.replace(_sc_params, needs_layout_passes=False)


def _deg_body(dst_hbm, zn_hbm, o_hbm, idx_v, hist_v, sem):
    cid = lax.axis_index("c")
    sid = lax.axis_index("s")
    wid = cid * 16 + sid
    pltpu.async_copy(dst_hbm.at[pl.ds(wid * EPT, EPT)], idx_v, sem).wait()
    pltpu.sync_copy(zn_hbm, hist_v)
    ones = jnp.full((16,), 1.0, jnp.float32)

    @pl.loop(0, EPT, step=64)
    def _(i):
        for u in range(4):
            idx = idx_v[pl.ds(i + u * 16, 16)]
            plsc.addupdate_scatter(hist_v, [idx], ones)

    pltpu.sync_copy(hist_v, o_hbm.at[wid])


def _sc_degree(dst_flat, zeros_n):
    kern = pl.kernel(
        _deg_body,
        out_type=jax.ShapeDtypeStruct((NTILES, NP), jnp.float32),
        mesh=_mesh,
        scratch_types=[
            pltpu.VMEM((EPT,), jnp.int32),
            pltpu.VMEM((NP,), jnp.float32),
            pltpu.SemaphoreType.DMA,
        ],
        compiler_params=_sc_params,
    )
    return kern(dst_flat, zeros_n)


def _scat_body(g_hbm, src_hbm, dst_hbm, zr_hbm, o_hbm,
               si0, si1, si2, di0, di1, di2, buf0, buf1, buf2, acc,
               qi0, qi1, qi2, ri0, ri1, ri2, sg0, sg1, sg2, ss0, ss1, ss2):
    cid = lax.axis_index("c")
    sid = lax.axis_index("s")
    wid = cid * 16 + sid
    base = wid * NCH
    sis = (si0, si1, si2)
    dis = (di0, di1, di2)
    bufs = (buf0, buf1, buf2)
    qis = (qi0, qi1, qi2)
    ris = (ri0, ri1, ri2)
    sgs = (sg0, sg1, sg2)
    sss = (ss0, ss1, ss2)

    for k in range(3):
        pltpu.async_copy(src_hbm.at[base + k], sis[k], qis[k])
        pltpu.async_copy(dst_hbm.at[base + k], dis[k], ris[k])
    pltpu.sync_copy(zr_hbm, acc.at[pl.ds(sid * STRIPE, STRIPE)])
    plsc.subcore_barrier()

    for k in range(3):
        pltpu.make_async_copy(src_hbm.at[base + k], sis[k], qis[k]).wait()
        pltpu.async_copy(g_hbm.at[sis[k]], bufs[k], sgs[k])

    @pl.loop(0, NCH - 3, step=3)
    def _(j):
        for k in range(3):
            pltpu.make_async_copy(g_hbm.at[sis[k]], bufs[k], sgs[k]).wait()
            pltpu.async_copy(src_hbm.at[base + j + k + 3], sis[k], qis[k])
            pltpu.make_async_copy(dst_hbm.at[base], dis[k], ris[k]).wait()
            pltpu.async_copy(bufs[k], acc.at[dis[k]], sss[k], add=True)
        for k in range(3):
            pltpu.make_async_copy(bufs[k], acc.at[dis[k]], sss[k]).wait()
            pltpu.async_copy(dst_hbm.at[base + j + k + 3], dis[k], ris[k])
            pltpu.make_async_copy(src_hbm.at[base], sis[k], qis[k]).wait()
            pltpu.async_copy(g_hbm.at[sis[k]], bufs[k], sgs[k])

    for k in range(3):
        pltpu.make_async_copy(g_hbm.at[sis[k]], bufs[k], sgs[k]).wait()
        pltpu.make_async_copy(dst_hbm.at[base], dis[k], ris[k]).wait()
        pltpu.async_copy(bufs[k], acc.at[dis[k]], sss[k], add=True)
    for k in range(3):
        pltpu.make_async_copy(bufs[k], acc.at[dis[k]], sss[k]).wait()
    plsc.subcore_barrier()

    pltpu.sync_copy(acc.at[pl.ds(sid * STRIPE, STRIPE)],
                    o_hbm.at[cid, pl.ds(sid * STRIPE, STRIPE)])


def _sc_scatter(g, src2d, dst2d, zrow):
    kern = pl.kernel(
        _scat_body,
        out_type=jax.ShapeDtypeStruct((2, NP, D), jnp.float32),
        mesh=_mesh,
        scratch_types=(
            [pltpu.VMEM((C,), jnp.int32)] * 6
            + [pltpu.VMEM((C, D), jnp.float32)] * 3
            + [pltpu.VMEM_SHARED((ACC_ROWS, D), jnp.float32)]
            + [pltpu.SemaphoreType.DMA] * 12
        ),
    )
    return kern(g, src2d, dst2d, zrow)


def _dinv_of(hist_ref):
    return lax.rsqrt(jnp.sum(hist_ref[...], axis=0) + 1.0)


def _tc1_body(hist_ref, x_ref, w_ref, g_ref):
    dinv = _dinv_of(hist_ref)
    m = jnp.dot(x_ref[...], w_ref[...], preferred_element_type=jnp.float32)
    g_ref[...] = m * dinv[:, None]


def _tc2_body(hist_ref, a_ref, g1_ref, b1_ref, w_ref, g2_ref):
    dinv = _dinv_of(hist_ref)
    h = dinv[:, None] * (a_ref[0] + a_ref[1] + g1_ref[...]) + b1_ref[...][None, :]
    h = jnp.maximum(h, 0.0)
    m = jnp.dot(h, w_ref[...], preferred_element_type=jnp.float32)
    g2_ref[...] = m * dinv[:, None]


def _tc3_body(hist_ref, a_ref, g2_ref, b2_ref, o_ref):
    dinv = _dinv_of(hist_ref)
    o_ref[...] = (dinv[:, None] * (a_ref[0] + a_ref[1] + g2_ref[...])
                  + b2_ref[...][None, :])


_HIST_SPEC = pl.BlockSpec((NTILES, BR), lambda i: (0, i))
_ROW_SPEC = pl.BlockSpec((BR, D), lambda i: (i, 0))
_ACC_SPEC = pl.BlockSpec((2, BR, D), lambda i: (0, i, 0))
_VEC_SPEC = pl.BlockSpec((D,), lambda i: (0,))
_W_SPEC = pl.BlockSpec((D, D), lambda i: (0, 0))
_OUT_ROW = jax.ShapeDtypeStruct((NP, D), jnp.float32)


def _tc1(hist, x_p, W1):
    return pl.pallas_call(
        _tc1_body,
        grid=(NP // BR,),
        in_specs=[_HIST_SPEC, _ROW_SPEC, _W_SPEC],
        out_specs=_ROW_SPEC,
        out_shape=_OUT_ROW,
    )(hist, x_p, W1)


def _tc2(hist, acc, g1, b1, W2):
    return pl.pallas_call(
        _tc2_body,
        grid=(NP // BR,),
        in_specs=[_HIST_SPEC, _ACC_SPEC, _ROW_SPEC, _VEC_SPEC, _W_SPEC],
        out_specs=_ROW_SPEC,
        out_shape=_OUT_ROW,
    )(hist, acc, g1, b1, W2)


def _tc3(hist, acc, g2, b2):
    return pl.pallas_call(
        _tc3_body,
        grid=(NP // BR,),
        in_specs=[_HIST_SPEC, _ACC_SPEC, _ROW_SPEC, _VEC_SPEC],
        out_specs=_ROW_SPEC,
        out_shape=_OUT_ROW,
    )(hist, acc, g2, b2)


def kernel(x, edge_index, W1, b1, W2, b2):
    src = edge_index[0]
    dst = edge_index[1]
    pad_i = jnp.arange(EP - E, dtype=jnp.int32)
    src_p = jnp.concatenate([src, pad_i % N])
    dst_p = jnp.concatenate([dst, N + pad_i % (ACC_ROWS - N)])
    src2d = src_p.reshape(NCH, NTILES, C).transpose(1, 0, 2).reshape(NTILES * NCH, C)
    dst2d = dst_p.reshape(NCH, NTILES, C).transpose(1, 0, 2).reshape(NTILES * NCH, C)
    x_p = jnp.concatenate([x, jnp.zeros((NP - N, D), jnp.float32)])
    zeros_n = jnp.zeros((NP,), jnp.float32)
    zrow = jnp.zeros((STRIPE, D), jnp.float32)

    hist = _sc_degree(dst_p, zeros_n)
    g1 = _tc1(hist, x_p, W1)
    acc1 = _sc_scatter(g1, src2d, dst2d, zrow)
    g2 = _tc2(hist, acc1, g1, b1, W2)
    acc2 = _sc_scatter(g2, src2d, dst2d, zrow)
    out = _tc3(hist, acc2, g2, b2)
    return out[:N]


# speedup vs baseline: 1.0139x; 1.0057x over previous
import dataclasses

import jax
import jax.numpy as jnp
from jax import lax
from jax.experimental import pallas as pl
from jax.experimental.pallas import tpu as pltpu
from jax.experimental.pallas import tpu_sc as plsc

N = 10000
E = 320000
D = 128

NP = 10240
BR = 2048
C = 128
NTILES = 32
NCH = 81
EPT = NCH * C
EP = NTILES * EPT
ACC_ROWS = 10112
STRIPE = ACC_ROWS // 16

_mesh = plsc.VectorSubcoreMesh(core_axis_name="c", subcore_axis_name="s")

_sc_params = pltpu.CompilerParams()
if "needs_layout_passes" in pltpu.CompilerParams.__dataclass_fields__:
    _sc_params = dataclasses.replace(_sc_params, needs_layout_passes=False)


def _deg_body(dst_hbm, zn_hbm, o_hbm, idx_v, hist_v, sem):
    cid = lax.axis_index("c")
    sid = lax.axis_index("s")
    wid = cid * 16 + sid
    pltpu.async_copy(dst_hbm.at[pl.ds(wid * EPT, EPT)], idx_v, sem).wait()
    pltpu.sync_copy(zn_hbm, hist_v)
    ones = jnp.full((16,), 1.0, jnp.float32)

    @pl.loop(0, EPT, step=64)
    def _(i):
        for u in range(4):
            idx = idx_v[pl.ds(i + u * 16, 16)]
            plsc.addupdate_scatter(hist_v, [idx], ones)

    pltpu.sync_copy(hist_v, o_hbm.at[wid])


def _sc_degree(dst_flat, zeros_n):
    kern = pl.kernel(
        _deg_body,
        out_type=jax.ShapeDtypeStruct((NTILES, NP), jnp.float32),
        mesh=_mesh,
        scratch_types=[
            pltpu.VMEM((EPT,), jnp.int32),
            pltpu.VMEM((NP,), jnp.float32),
            pltpu.SemaphoreType.DMA,
        ],
        compiler_params=_sc_params,
    )
    return kern(dst_flat, zeros_n)


def _scat_body(g_hbm, src_hbm, dst_hbm, zr_hbm, o_hbm,
               si0, si1, si2, di0, di1, di2, buf0, buf1, buf2, acc,
               qi0, qi1, qi2, ri0, ri1, ri2, sg0, sg1, sg2, ss0, ss1, ss2):
    cid = lax.axis_index("c")
    sid = lax.axis_index("s")
    wid = cid * 16 + sid
    base = wid * NCH
    sis = (si0, si1, si2)
    dis = (di0, di1, di2)
    bufs = (buf0, buf1, buf2)
    qis = (qi0, qi1, qi2)
    ris = (ri0, ri1, ri2)
    sgs = (sg0, sg1, sg2)
    sss = (ss0, ss1, ss2)

    for k in range(3):
        pltpu.async_copy(src_hbm.at[base + k], sis[k], qis[k])
        pltpu.async_copy(dst_hbm.at[base + k], dis[k], ris[k])
    pltpu.sync_copy(zr_hbm, acc.at[pl.ds(sid * STRIPE, STRIPE)])

    for k in range(3):
        pltpu.make_async_copy(src_hbm.at[base + k], sis[k], qis[k]).wait()
        pltpu.async_copy(g_hbm.at[sis[k]], bufs[k], sgs[k])
    plsc.subcore_barrier()

    @pl.loop(0, NCH - 3, step=3)
    def _(j):
        for k in range(3):
            pltpu.make_async_copy(g_hbm.at[sis[k]], bufs[k], sgs[k]).wait()
            pltpu.async_copy(src_hbm.at[base + j + k + 3], sis[k], qis[k])
            pltpu.make_async_copy(dst_hbm.at[base], dis[k], ris[k]).wait()
            pltpu.async_copy(bufs[k], acc.at[dis[k]], sss[k], add=True)
        for k in range(3):
            pltpu.make_async_copy(bufs[k], acc.at[dis[k]], sss[k]).wait()
            pltpu.async_copy(dst_hbm.at[base + j + k + 3], dis[k], ris[k])
            pltpu.make_async_copy(src_hbm.at[base], sis[k], qis[k]).wait()
            pltpu.async_copy(g_hbm.at[sis[k]], bufs[k], sgs[k])

    for k in range(3):
        pltpu.make_async_copy(g_hbm.at[sis[k]], bufs[k], sgs[k]).wait()
        pltpu.make_async_copy(dst_hbm.at[base], dis[k], ris[k]).wait()
        pltpu.async_copy(bufs[k], acc.at[dis[k]], sss[k], add=True)
    for k in range(3):
        pltpu.make_async_copy(bufs[k], acc.at[dis[k]], sss[k]).wait()
    plsc.subcore_barrier()

    pltpu.sync_copy(acc.at[pl.ds(sid * STRIPE, STRIPE)],
                    o_hbm.at[cid, pl.ds(sid * STRIPE, STRIPE)])


def _sc_scatter(g, src2d, dst2d, zrow):
    kern = pl.kernel(
        _scat_body,
        out_type=jax.ShapeDtypeStruct((2, NP, D), jnp.float32),
        mesh=_mesh,
        scratch_types=(
            [pltpu.VMEM((C,), jnp.int32)] * 6
            + [pltpu.VMEM((C, D), jnp.float32)] * 3
            + [pltpu.VMEM_SHARED((ACC_ROWS, D), jnp.float32)]
            + [pltpu.SemaphoreType.DMA] * 12
        ),
    )
    return kern(g, src2d, dst2d, zrow)


def _dinv_of(hist_ref):
    return lax.rsqrt(jnp.sum(hist_ref[...], axis=0) + 1.0)


def _tc1_body(hist_ref, x_ref, w_ref, g_ref):
    dinv = _dinv_of(hist_ref)
    m = jnp.dot(x_ref[...], w_ref[...], preferred_element_type=jnp.float32)
    g_ref[...] = m * dinv[:, None]


def _tc2_body(hist_ref, a_ref, g1_ref, b1_ref, w_ref, g2_ref):
    dinv = _dinv_of(hist_ref)
    h = dinv[:, None] * (a_ref[0] + a_ref[1] + g1_ref[...]) + b1_ref[...][None, :]
    h = jnp.maximum(h, 0.0)
    m = jnp.dot(h, w_ref[...], preferred_element_type=jnp.float32)
    g2_ref[...] = m * dinv[:, None]


def _tc3_body(hist_ref, a_ref, g2_ref, b2_ref, o_ref):
    dinv = _dinv_of(hist_ref)
    o_ref[...] = (dinv[:, None] * (a_ref[0] + a_ref[1] + g2_ref[...])
                  + b2_ref[...][None, :])


_HIST_SPEC = pl.BlockSpec((NTILES, BR), lambda i: (0, i))
_ROW_SPEC = pl.BlockSpec((BR, D), lambda i: (i, 0))
_ACC_SPEC = pl.BlockSpec((2, BR, D), lambda i: (0, i, 0))
_VEC_SPEC = pl.BlockSpec((D,), lambda i: (0,))
_W_SPEC = pl.BlockSpec((D, D), lambda i: (0, 0))
_OUT_ROW = jax.ShapeDtypeStruct((NP, D), jnp.float32)


def _tc1(hist, x_p, W1):
    return pl.pallas_call(
        _tc1_body,
        grid=(NP // BR,),
        in_specs=[_HIST_SPEC, _ROW_SPEC, _W_SPEC],
        out_specs=_ROW_SPEC,
        out_shape=_OUT_ROW,
    )(hist, x_p, W1)


def _tc2(hist, acc, g1, b1, W2):
    return pl.pallas_call(
        _tc2_body,
        grid=(NP // BR,),
        in_specs=[_HIST_SPEC, _ACC_SPEC, _ROW_SPEC, _VEC_SPEC, _W_SPEC],
        out_specs=_ROW_SPEC,
        out_shape=_OUT_ROW,
    )(hist, acc, g1, b1, W2)


def _tc3(hist, acc, g2, b2):
    return pl.pallas_call(
        _tc3_body,
        grid=(NP // BR,),
        in_specs=[_HIST_SPEC, _ACC_SPEC, _ROW_SPEC, _VEC_SPEC],
        out_specs=_ROW_SPEC,
        out_shape=_OUT_ROW,
    )(hist, acc, g2, b2)


def kernel(x, edge_index, W1, b1, W2, b2):
    src = edge_index[0]
    dst = edge_index[1]
    pad_i = jnp.arange(EP - E, dtype=jnp.int32)
    src_p = jnp.concatenate([src, pad_i % N])
    dst_p = jnp.concatenate([dst, N + pad_i % (ACC_ROWS - N)])
    src2d = src_p.reshape(NCH, NTILES, C).transpose(1, 0, 2).reshape(NTILES * NCH, C)
    dst2d = dst_p.reshape(NCH, NTILES, C).transpose(1, 0, 2).reshape(NTILES * NCH, C)
    x_p = jnp.concatenate([x, jnp.zeros((NP - N, D), jnp.float32)])
    zeros_n = jnp.zeros((NP,), jnp.float32)
    zrow = jnp.zeros((STRIPE, D), jnp.float32)

    hist = _sc_degree(dst_p, zeros_n)
    g1 = _tc1(hist, x_p, W1)
    acc1 = _sc_scatter(g1, src2d, dst2d, zrow)
    g2 = _tc2(hist, acc1, g1, b1, W2)
    acc2 = _sc_scatter(g2, src2d, dst2d, zrow)
    out = _tc3(hist, acc2, g2, b2)
    return out[:N]
